# baseline (speedup 1.0000x reference)
import functools

import jax
import jax.numpy as jnp
from jax import lax
from jax.experimental import pallas as pl
from jax.experimental.pallas import tpu as pltpu
from jax.experimental.pallas import tpu_sc as plsc

_B, _C, _Hp, _Wp = 4, 96, 192, 192
_H, _W = 384, 384

_B_TC = 3
_B_SC = _B - _B_TC
_N_SC = _B_SC * _C

_NC, _NS, _L = 2, 16, 16
_NW = _NC * _NS
_PPW = _N_SC // _NW

_CH = 32
_VPR = _Wp // _L
_CHUNKS = _Hp // _CH
_T = _PPW * _CHUNKS
_OCH = 2 * _CH


def _sc_body(x_hbm, idx_hbm, out_hbm,
             xv0, xv1, iv0, iv1, rv0, rv1, ov0, ov1,
             sin0, sin1, sout0, sout1):
    wid = lax.axis_index("s") * _NC + lax.axis_index("c")
    xv = (xv0, xv1)
    iv = (iv0, iv1)
    rv = (rv0, rv1)
    ov = (ov0, ov1)
    sin = (sin0, sin1)
    sout = (sout0, sout1)

    zeros16 = jnp.zeros((_L,), jnp.float32)

    def coords(t):
        plane = wid * _PPW + t // _CHUNKS
        b = _B_TC + plane // _C
        c = plane % _C
        i0 = (t % _CHUNKS) * _CH
        return b, c, i0

    def start_in(t, buf):
        b, c, i0 = coords(t)
        pltpu.async_copy(x_hbm.at[b, c, pl.ds(i0, _CH), :], xv[buf], sin[buf])
        pltpu.async_copy(idx_hbm.at[b, c, pl.ds(i0, _CH), :], iv[buf], sin[buf])

    def wait_in(buf):
        pltpu.make_async_copy(x_hbm.at[0, 0, pl.ds(0, _CH), :], xv[buf],
                              sin[buf]).wait()
        pltpu.make_async_copy(idx_hbm.at[0, 0, pl.ds(0, _CH), :], iv[buf],
                              sin[buf]).wait()

    def wait_out(buf):
        pltpu.make_async_copy(ov[buf], out_hbm.at[0, pl.ds(0, _OCH), :],
                              sout[buf]).wait()

    @plsc.parallel_loop(0, _OCH, step=1, unroll=2)
    def _(r):
        for v in range(_W // _L):
            ov0[r, pl.ds(v * _L, _L)] = zeros16
            ov1[r, pl.ds(v * _L, _L)] = zeros16

    start_in(0, 0)

    def step_pair(i, carry):
        for buf in range(2):
            t = 2 * i + buf

            @pl.when(i >= 1)
            def _():
                wait_out(buf)

                @plsc.parallel_loop(0, _CH, step=1, unroll=2)
                def _(k):
                    row0 = 2 * k
                    for v in range(_VPR):
                        t1 = rv[buf][k, pl.ds(v * _L, _L)]
                        di = t1 >= _W
                        col = jnp.where(di, t1 - _W, t1)
                        row = jnp.where(di, row0 + 1, row0)
                        plsc.store_scatter(ov[buf], [row, col], zeros16)

            wait_in(buf)

            if buf == 0:
                start_in(t + 1, 1)
            else:
                @pl.when(i < _T // 2 - 1)
                def _():
                    start_in(t + 1, 0)

            _, _, i0 = coords(t)

            @plsc.parallel_loop(0, _CH, step=1, unroll=2)
            def _(k):
                base = 768 * (i0 + k)
                row0 = 2 * k
                for v in range(_VPR):
                    xvec = xv[buf][k, pl.ds(v * _L, _L)]
                    ivec = iv[buf][k, pl.ds(v * _L, _L)]
                    t1 = ivec - base
                    rv[buf][k, pl.ds(v * _L, _L)] = t1
                    di = t1 >= _W
                    col = jnp.where(di, t1 - _W, t1)
                    row = jnp.where(di, row0 + 1, row0)
                    plsc.store_scatter(ov[buf], [row, col], xvec)

            b, c, i0 = coords(t)
            pltpu.async_copy(
                ov[buf],
                out_hbm.at[(b - _B_TC) * _C + c, pl.ds(2 * i0, _OCH), :],
                sout[buf])
        return carry

    lax.fori_loop(0, _T // 2, step_pair, 0)

    wait_out(0)
    wait_out(1)


def _tc_body(x_ref, idx_ref, out_ref):
    x = x_ref[0, 0]
    idx = idx_ref[0, 0]
    ivec = lax.broadcasted_iota(jnp.int32, (_Hp, _Wp), 0)
    rem = idx - 768 * ivec
    di1 = rem >= _W
    dj1 = (idx & 1) == 1

    xb = x.astype(jnp.bfloat16)
    zero = jnp.zeros_like(xb)
    a00 = jnp.where(jnp.logical_and(~di1, ~dj1), xb, zero)
    a01 = jnp.where(jnp.logical_and(~di1, dj1), xb, zero)
    a10 = jnp.where(jnp.logical_and(di1, ~dj1), xb, zero)
    a11 = jnp.where(jnp.logical_and(di1, dj1), xb, zero)

    jcol = lax.broadcasted_iota(jnp.int32, (_Wp, _W), 0)
    ocol = lax.broadcasted_iota(jnp.int32, (_Wp, _W), 1)
    E = (ocol == 2 * jcol).astype(jnp.bfloat16)
    O = (ocol == 2 * jcol + 1).astype(jnp.bfloat16)

    dot = functools.partial(
        lax.dot_general,
        dimension_numbers=(((1,), (0,)), ((), ())),
        preferred_element_type=jnp.float32,
    )
    row_even = dot(a00, E) + dot(a01, O)
    row_odd = dot(a10, E) + dot(a11, O)

    orow = lax.broadcasted_iota(jnp.int32, (_H, _Hp), 0)
    irow = lax.broadcasted_iota(jnp.int32, (_H, _Hp), 1)
    P0 = (orow == 2 * irow).astype(jnp.bfloat16)
    P1 = (orow == 2 * irow + 1).astype(jnp.bfloat16)

    out_ref[0, 0] = dot(P0, row_even.astype(jnp.bfloat16)) + dot(
        P1, row_odd.astype(jnp.bfloat16))


def _tc_unpool(x, idx):
    return pl.pallas_call(
        _tc_body,
        grid=(_B_TC * _C,),
        in_specs=[
            pl.BlockSpec((1, 1, _Hp, _Wp), lambda p: (p // _C, p % _C, 0, 0)),
            pl.BlockSpec((1, 1, _Hp, _Wp), lambda p: (p // _C, p % _C, 0, 0)),
        ],
        out_specs=pl.BlockSpec((1, 1, _H, _W), lambda p: (p // _C, p % _C, 0, 0)),
        out_shape=jax.ShapeDtypeStruct((_B_TC, _C, _H, _W), jnp.float32),
    )(x, idx)


def _sc_unpool(x, indices):
    mesh = plsc.VectorSubcoreMesh(core_axis_name="c", subcore_axis_name="s")
    run = functools.partial(
        pl.kernel,
        mesh=mesh,
        out_type=jax.ShapeDtypeStruct((_N_SC, _H, _W), jnp.float32),
        compiler_params=pltpu.CompilerParams(needs_layout_passes=False),
        scratch_types=[
            pltpu.VMEM((_CH, _Wp), jnp.float32),
            pltpu.VMEM((_CH, _Wp), jnp.float32),
            pltpu.VMEM((_CH, _Wp), jnp.int32),
            pltpu.VMEM((_CH, _Wp), jnp.int32),
            pltpu.VMEM((_CH, _Wp), jnp.int32),
            pltpu.VMEM((_CH, _Wp), jnp.int32),
            pltpu.VMEM((_OCH, _W), jnp.float32),
            pltpu.VMEM((_OCH, _W), jnp.float32),
            pltpu.SemaphoreType.DMA,
            pltpu.SemaphoreType.DMA,
            pltpu.SemaphoreType.DMA,
            pltpu.SemaphoreType.DMA,
        ],
    )(_sc_body)
    return run(x, indices)


@jax.jit
def _unpool(x, indices):
    sc_out = _sc_unpool(x, indices)
    tc_out = _tc_unpool(x, indices)
    return jnp.concatenate(
        [tc_out, sc_out.reshape(_B_SC, _C, _H, _W)], axis=0)


def kernel(x, indices):
    return _unpool(x, indices)
